# baseline (speedup 1.0000x reference)
import jax
import jax.numpy as jnp
from jax.experimental import pallas as pl

H = 128
EPS = 1e-5
REL = [('vv', 'v', 'v'), ('ve', 'v', 'e'), ('vf', 'v', 'f'), ('ev', 'e', 'v'),
       ('ef', 'e', 'f'), ('fv', 'f', 'v'), ('fe', 'f', 'e')]


def _agg(x_src, ei, n_dst):
    src, dst = ei[0], ei[1]
    s = jax.ops.segment_sum(x_src[src], dst, num_segments=n_dst)
    c = jax.ops.segment_sum(jnp.ones((ei.shape[1],), jnp.float32), dst,
                            num_segments=n_dst)
    return s / jnp.maximum(c, 1.0)[:, None]


def _sage(x_src, x_dst, ei, Wl, bl, Wr):
    return _agg(x_src, ei, x_dst.shape[0]) @ Wl + bl + x_dst @ Wr


def _bn(x, g, b):
    return x / jnp.sqrt(1.0 + EPS) * g + b


def _hetero_layer(xd, eid, p, lname):
    acc = {'v': [], 'e': [], 'f': []}
    for name, st, dt in REL:
        o = _sage(xd[st], xd[dt], eid[name], p[lname + '_' + name + '_Wl'],
                  p[lname + '_' + name + '_bl'], p[lname + '_' + name + '_Wr'])
        acc[dt].append(o)
    return {t: jnp.mean(jnp.stack(acc[t]), axis=0) for t in acc}


def _head_body(pooled_ref, w1_ref, b1_ref, g_ref, bb_ref, w2_ref, b2_ref,
               out_ref):
    h = pooled_ref[...] @ w1_ref[...] + b1_ref[...]
    h = jax.nn.relu(h * (1.0 / jnp.sqrt(1.0 + EPS)) * g_ref[...] + bb_ref[...])
    out_ref[...] = h @ w2_ref[...] + b2_ref[...]


def _head(pooled, p):
    out = pl.pallas_call(
        _head_body,
        out_shape=jax.ShapeDtypeStruct((1, 10), jnp.float32),
    )(pooled[None, :], p['fc1_W'], p['fc1_b'][None, :], p['fcbn_g'][None, :],
      p['fcbn_b'][None, :], p['fc2_W'], p['fc2_b'][None, :])
    return out[0]


def kernel(x_v, x_e, x_f, ei_vv, ei_ve, ei_vf, ei_ev, ei_ef, ei_fv, ei_fe,
           params):
    p = params
    eid = {'vv': ei_vv, 've': ei_ve, 'vf': ei_vf, 'ev': ei_ev, 'ef': ei_ef,
           'fv': ei_fv, 'fe': ei_fe}
    xd = {'v': x_v, 'e': x_e, 'f': x_f}
    conv = _hetero_layer(xd, eid, p, 'l0')
    xd = {t: jax.nn.relu(_bn(conv[t], p['bn_l0_' + t + '_g'],
                             p['bn_l0_' + t + '_b'])) for t in conv}
    orig = xd
    conv = _hetero_layer(xd, eid, p, 'l1')
    xd = {t: jax.nn.relu(_bn(conv[t] + orig[t], p['bn_l1_' + t + '_g'],
                             p['bn_l1_' + t + '_b'])) for t in conv}
    pooled = jnp.concatenate([xd['v'].mean(axis=0), xd['e'].mean(axis=0),
                              xd['f'].mean(axis=0)], axis=0)
    return _head(pooled, p)


# speedup vs baseline: 1.9305x; 1.9305x over previous
import functools

import jax
import jax.numpy as jnp
from jax import lax
from jax.experimental import pallas as pl
from jax.experimental.pallas import tpu as pltpu
from jax.experimental.pallas import tpu_sc as plsc

H = 128
EPS = 1e-5
NV, NE, NF = 20000, 60000, 40000
E = 128000
EPAD = 131072
REL = [('vv', 'v', 'v'), ('ve', 'v', 'e'), ('vf', 'v', 'f'), ('ev', 'e', 'v'),
       ('ef', 'e', 'f'), ('fv', 'f', 'v'), ('fe', 'f', 'e')]
FEAT0 = {'v': 7, 'e': 2, 'f': 5}
NN_T = {'v': NV, 'e': NE, 'f': NF}
OFF_T = {'v': 0, 'e': NV, 'f': NV + NE}
RELS_BY_DST = {'v': [0, 3, 5], 'e': [1, 6], 'f': [2, 4]}
N_DT = [20000, 60000, 40000, 20000, 40000, 20000, 60000]
RELBASE = [0, 20000, 80000, 120000, 140000, 180000, 200000]
TOT0 = 260000
BS = 12800
ACC_ROWS = BS + 64
BN = 2000

PASSES = []
for _r in range(7):
    _n = N_DT[_r]
    for _b in range((_n + BS - 1) // BS):
        PASSES.append((_r, _b, min(BS, _n - _b * BS)))
NPASS = len(PASSES)
REL_OF_PASS = [t[0] for t in PASSES]
ROWS_OF_PASS = [t[2] for t in PASSES]
BASE_OF_PASS = [RELBASE[t[0]] + BS * t[1] for t in PASSES]
NZL_OF_PASS = [(r + 63) // 64 for r in ROWS_OF_PASS]
NDM_OF_PASS = [r // 800 for r in ROWS_OF_PASS]


def _sel(p, vals):
    out = jnp.int32(vals[-1])
    for i in range(len(vals) - 2, -1, -1):
        out = jnp.where(p == i, jnp.int32(vals[i]), out)
    return out


def _make_sc_agg():
    mesh = plsc.VectorSubcoreMesh(core_axis_name="c", subcore_axis_name="s")

    @functools.partial(
        pl.kernel,
        out_type=jax.ShapeDtypeStruct((TOT0, H), jnp.float32),
        mesh=mesh,
        scratch_types=[
            pltpu.VMEM((8, 128), jnp.int32),
            pltpu.VMEM((8, 128), jnp.int32),
            pltpu.VMEM((128, H), jnp.float32),
            pltpu.VMEM((64, H), jnp.float32),
            pltpu.VMEM_SHARED((ACC_ROWS, H), jnp.float32),
            pltpu.SemaphoreType.DMA,
        ],
    )
    def sc_agg(tbl, sidx, didx, zeros_h, out, sidx_v, didx_v, rows_v, zbuf,
               acc, sem):
        c = lax.axis_index("c")
        s = lax.axis_index("s")
        pltpu.sync_copy(zeros_h, zbuf)

        def pass_body(p, carry):
            rel = _sel(p, REL_OF_PASS)
            out_base = _sel(p, BASE_OF_PASS)
            nzl = _sel(p, NZL_OF_PASS)
            ndm = _sel(p, NDM_OF_PASS)

            def zero_body(i, carry2):
                k = s + 16 * i

                @pl.when(k < nzl)
                def _():
                    pltpu.sync_copy(
                        zbuf, acc.at[pl.ds(pl.multiple_of(k * 64, 8), 64)])

                return carry2

            lax.fori_loop(0, 13, zero_body, None)
            plsc.subcore_barrier()

            def macro_body(m, carry2):
                row0 = pl.multiple_of(s * 64 + m * 8, 8)
                pltpu.sync_copy(sidx.at[rel, pl.ds(row0, 8)], sidx_v)
                pltpu.sync_copy(didx.at[p, pl.ds(row0, 8)], didx_v)

                def micro_body(j, carry3):
                    pltpu.async_copy(tbl.at[sidx_v.at[j]], rows_v, sem).wait()
                    pltpu.sync_copy(rows_v, acc.at[didx_v.at[j]], add=True)
                    return carry3

                lax.fori_loop(0, 8, micro_body, None)
                return carry2

            lax.fori_loop(0, 8, macro_body, None)
            plsc.subcore_barrier()

            def dump_body(i, carry2):
                k = s + 16 * i

                @pl.when(k < ndm)
                def _():
                    pltpu.sync_copy(
                        acc.at[pl.ds(pl.multiple_of(k * 800, 8), 800)],
                        out.at[pl.ds(
                            pl.multiple_of(out_base + k * 800, 8), 800)])

                return carry2

            lax.fori_loop(0, 1, dump_body, None)
            plsc.subcore_barrier()
            return carry

        lo = jnp.where(c == 0, 0, NPASS // 2)
        hi = jnp.where(c == 0, NPASS // 2, NPASS)
        lax.fori_loop(lo, hi, pass_body, None)

    return sc_agg


_sc_agg = _make_sc_agg()


def _l0_combine(sum0, x128, wcat, t):
    n_t = NN_T[t]
    rels = RELS_BY_DST[t]
    k = len(rels)

    def body(*refs):
        sum_refs = refs[:k]
        x_ref, w_ref, o_ref = refs[k], refs[k + 1], refs[k + 2]
        parts = []
        for sr_ref in sum_refs:
            sr = sr_ref[:, :16]
            invc = 1.0 / jnp.maximum(sr[:, 15:16], 1.0)
            parts.append(sr * invc)
        parts.append(x_ref[:, :16])
        a = jnp.concatenate(parts, axis=1)
        o_ref[...] = jnp.maximum(
            jax.lax.dot(a, w_ref[...], preferred_element_type=jnp.float32),
            0.0)

    in_specs = [
        pl.BlockSpec((BN, H), (lambda i, b=RELBASE[r] // BN: (b + i, 0)))
        for r in rels
    ]
    in_specs.append(
        pl.BlockSpec((BN, H), (lambda i, b=OFF_T[t] // BN: (b + i, 0))))
    in_specs.append(pl.BlockSpec((16 * (k + 1), H), lambda i: (0, 0)))
    return pl.pallas_call(
        body,
        grid=(n_t // BN,),
        in_specs=in_specs,
        out_specs=pl.BlockSpec((BN, H), lambda i: (i, 0)),
        out_shape=jax.ShapeDtypeStruct((n_t, H), jnp.float32),
    )(*([sum0] * k + [x128, wcat]))


def _l1_combine(sum1, sum0, ht, wls, m1, beta, t):
    n_t = NN_T[t]
    rels = RELS_BY_DST[t]
    k = len(rels)

    def body(*refs):
        agg_refs = refs[:k]
        cnt_refs = refs[k:2 * k]
        h_ref = refs[2 * k]
        wl_refs = refs[2 * k + 1:3 * k + 1]
        m1_ref, beta_ref, o_ref = refs[3 * k + 1:3 * k + 4]
        acc = jax.lax.dot(h_ref[...], m1_ref[...],
                          preferred_element_type=jnp.float32)
        for ri in range(k):
            invc = 1.0 / jnp.maximum(cnt_refs[ri][:, 15:16], 1.0)
            acc = acc + jax.lax.dot(agg_refs[ri][...] * invc, wl_refs[ri][...],
                                    preferred_element_type=jnp.float32)
        x = jnp.maximum(acc + beta_ref[...], 0.0)

        @pl.when(pl.program_id(0) == 0)
        def _():
            o_ref[...] = jnp.zeros_like(o_ref)

        o_ref[...] += jnp.sum(x, axis=0, keepdims=True)

    in_specs = []
    args = []
    for r in rels:
        in_specs.append(
            pl.BlockSpec((BN, H), (lambda i, b=RELBASE[r] // BN: (b + i, 0))))
        args.append(sum1)
    for r in rels:
        in_specs.append(
            pl.BlockSpec((BN, H), (lambda i, b=RELBASE[r] // BN: (b + i, 0))))
        args.append(sum0)
    in_specs.append(
        pl.BlockSpec((BN, H), (lambda i, b=OFF_T[t] // BN: (b + i, 0))))
    args.append(ht)
    for ri in range(k):
        in_specs.append(pl.BlockSpec((H, H), lambda i: (0, 0)))
        args.append(wls[ri])
    in_specs.append(pl.BlockSpec((H, H), lambda i: (0, 0)))
    args.append(m1)
    in_specs.append(pl.BlockSpec((1, H), lambda i: (0, 0)))
    args.append(beta)
    return pl.pallas_call(
        body,
        grid=(n_t // BN,),
        in_specs=in_specs,
        out_specs=pl.BlockSpec((1, H), lambda i: (0, 0)),
        out_shape=jax.ShapeDtypeStruct((1, H), jnp.float32),
    )(*args)


def _head(pools, w1, b1, w2, b2):
    def body(pv, pe, pf, w1r, b1r, w2r, b2r, o_ref):
        pooled = jnp.concatenate([pv[...], pe[...], pf[...]], axis=1)
        h = jnp.maximum(
            jax.lax.dot(pooled, w1r[...], preferred_element_type=jnp.float32)
            + b1r[...], 0.0)
        o_ref[...] = jax.lax.dot(
            h, w2r[...], preferred_element_type=jnp.float32) + b2r[...]

    return pl.pallas_call(
        body,
        out_shape=jax.ShapeDtypeStruct((1, 128), jnp.float32),
    )(pools[0], pools[1], pools[2], w1, b1, w2, b2)


def kernel(x_v, x_e, x_f, ei_vv, ei_ve, ei_vf, ei_ev, ei_ef, ei_fv, ei_fe,
           params):
    p = params
    eid = {'vv': ei_vv, 've': ei_ve, 'vf': ei_vf, 'ev': ei_ev, 'ef': ei_ef,
           'fv': ei_fv, 'fe': ei_fe}
    s = (1.0 + EPS) ** -0.5

    npad = EPAD - E
    pad_src = jnp.arange(npad, dtype=jnp.int32) % 128
    dummy = BS + (jnp.arange(EPAD, dtype=jnp.int32) % 64)
    sidx = jnp.stack([
        jnp.concatenate([eid[name][0] + OFF_T[st], pad_src])
        for name, st, dt in REL
    ]).reshape(7, 1024, 128)
    dst_pad = {
        r: jnp.concatenate(
            [eid[REL[r][0]][1],
             jnp.full((npad,), -1, jnp.int32)]) for r in range(7)
    }
    didx = jnp.stack([
        jnp.where((dst_pad[r] >= b * BS) & (dst_pad[r] < b * BS + BS),
                  dst_pad[r] - b * BS, dummy) for r, b, _rows in PASSES
    ]).reshape(NPASS, 1024, 128)

    def pad128(x):
        n, f = x.shape
        return jnp.concatenate(
            [x, jnp.zeros((n, 15 - f), jnp.float32),
             jnp.ones((n, 1), jnp.float32),
             jnp.zeros((n, 112), jnp.float32)], axis=1)

    x128 = jnp.concatenate([pad128(x_v), pad128(x_e), pad128(x_f)], axis=0)

    wcat0 = {}
    wl1p = {}
    m1f = {}
    beta1 = {}
    for t in ('v', 'e', 'f'):
        g0 = p['bn_l0_' + t + '_g']
        b0 = p['bn_l0_' + t + '_b']
        rels = RELS_BY_DST[t]
        K = len(rels)
        ft = FEAT0[t]
        parts = []
        for r in rels:
            name, st, _ = REL[r]
            wl = p['l0_' + name + '_Wl']
            wl16 = jnp.zeros((16, H), jnp.float32).at[:FEAT0[st]].set(wl)
            parts.append(wl16 * (s / K) * g0[None, :])
        wrm = sum(p['l0_' + REL[r][0] + '_Wr'] for r in rels) / K
        blm = sum(p['l0_' + REL[r][0] + '_bl'] for r in rels) / K
        m0 = jnp.zeros((16, H), jnp.float32)
        m0 = m0.at[:ft].set(s * wrm * g0[None, :])
        m0 = m0.at[15].set(s * blm * g0 + b0)
        wcat0[t] = jnp.concatenate(parts + [m0], axis=0)

        g1 = p['bn_l1_' + t + '_g']
        b1 = p['bn_l1_' + t + '_b']
        wl1p[t] = [
            p['l1_' + REL[r][0] + '_Wl'] * (s / K) * g1[None, :] for r in rels
        ]
        wr1m = sum(p['l1_' + REL[r][0] + '_Wr'] for r in rels) / K
        m1f[t] = s * (wr1m + jnp.eye(H, dtype=jnp.float32)) * g1[None, :]
        bl1m = sum(p['l1_' + REL[r][0] + '_bl'] for r in rels) / K
        beta1[t] = (s * bl1m * g1 + b1)[None, :]

    gf = p['fcbn_g']
    bf = p['fcbn_b']
    rowscale = jnp.concatenate([
        jnp.full((H,), 1.0 / NV, jnp.float32),
        jnp.full((H,), 1.0 / NE, jnp.float32),
        jnp.full((H,), 1.0 / NF, jnp.float32)])
    w1f = p['fc1_W'] * rowscale[:, None] * (s * gf)[None, :]
    b1f = (s * p['fc1_b'] * gf + bf)[None, :]
    w2p = jnp.zeros((256, 128), jnp.float32).at[:, :10].set(p['fc2_W'])
    b2p = jnp.zeros((1, 128), jnp.float32).at[0, :10].set(p['fc2_b'])

    zeros_h = jnp.zeros((64, H), jnp.float32)

    x128, sidx, didx, zeros_h = lax.optimization_barrier(
        (x128, sidx, didx, zeros_h))

    sum0 = _sc_agg(x128, sidx, didx, zeros_h)
    h0 = {t: _l0_combine(sum0, x128, wcat0[t], t) for t in ('v', 'e', 'f')}
    ht = jnp.concatenate([h0['v'], h0['e'], h0['f']], axis=0)

    ht = lax.optimization_barrier(ht)
    sum1 = _sc_agg(ht, sidx, didx, zeros_h)
    pools = [
        _l1_combine(sum1, sum0, ht, wl1p[t], m1f[t], beta1[t], t)
        for t in ('v', 'e', 'f')
    ]

    out = _head(pools, w1f, b1f, w2p, b2p)
    return out[0, :10]


# speedup vs baseline: 2.0741x; 1.0744x over previous
import functools

import jax
import jax.numpy as jnp
from jax import lax
from jax.experimental import pallas as pl
from jax.experimental.pallas import tpu as pltpu
from jax.experimental.pallas import tpu_sc as plsc

H = 128
EPS = 1e-5
NV, NE, NF = 20000, 60000, 40000
E = 128000
EPAD = 131072
REL = [('vv', 'v', 'v'), ('ve', 'v', 'e'), ('vf', 'v', 'f'), ('ev', 'e', 'v'),
       ('ef', 'e', 'f'), ('fv', 'f', 'v'), ('fe', 'f', 'e')]
FEAT0 = {'v': 7, 'e': 2, 'f': 5}
NN_T = {'v': NV, 'e': NE, 'f': NF}
OFF_T = {'v': 0, 'e': NV, 'f': NV + NE}
RELS_BY_DST = {'v': [0, 3, 5], 'e': [1, 6], 'f': [2, 4]}
N_DT = [20000, 60000, 40000, 20000, 40000, 20000, 60000]
RELBASE = [0, 20000, 80000, 120000, 140000, 180000, 200000]
TOT0 = 260000
BS = 10400
ACC_ROWS = BS + 64
BN = 2000

PASSES = []
for _r in range(7):
    _n = N_DT[_r]
    for _b in range((_n + BS - 1) // BS):
        PASSES.append((_r, _b, min(BS, _n - _b * BS)))
NPASS = len(PASSES)
REL_OF_PASS = [t[0] for t in PASSES]
ROWS_OF_PASS = [t[2] for t in PASSES]
BASE_OF_PASS = [RELBASE[t[0]] + BS * t[1] for t in PASSES]
NZL_OF_PASS = [(r + 63) // 64 for r in ROWS_OF_PASS]
NDM_OF_PASS = [r // 800 for r in ROWS_OF_PASS]


def _sel(p, vals):
    out = jnp.int32(vals[-1])
    for i in range(len(vals) - 2, -1, -1):
        out = jnp.where(p == i, jnp.int32(vals[i]), out)
    return out


def _make_sc_agg():
    mesh = plsc.VectorSubcoreMesh(core_axis_name="c", subcore_axis_name="s")

    @functools.partial(
        pl.kernel,
        out_type=jax.ShapeDtypeStruct((TOT0, H), jnp.float32),
        mesh=mesh,
        scratch_types=[
            pltpu.VMEM((8, 128), jnp.int32),
            pltpu.VMEM((8, 128), jnp.int32),
            pltpu.VMEM((256, H), jnp.float32),
            pltpu.VMEM((64, H), jnp.float32),
            pltpu.VMEM_SHARED((ACC_ROWS, H), jnp.float32),
            pltpu.SemaphoreType.DMA,
            pltpu.SemaphoreType.DMA,
            pltpu.SemaphoreType.DMA,
            pltpu.SemaphoreType.DMA,
        ],
    )
    def sc_agg(tbl, sidx, didx, zeros_h, out, sidx_v, didx_v, rows_v, zbuf,
               acc, sem_ga, sem_gb, sem_sa, sem_sb):
        c = lax.axis_index("c")
        s = lax.axis_index("s")
        pltpu.sync_copy(zeros_h, zbuf)

        def pass_body(p, carry):
            rel = _sel(p, REL_OF_PASS)
            out_base = _sel(p, BASE_OF_PASS)
            nzl = _sel(p, NZL_OF_PASS)
            ndm = _sel(p, NDM_OF_PASS)

            def zero_body(i, carry2):
                k = s + 16 * i

                @pl.when(k < nzl)
                def _():
                    pltpu.sync_copy(
                        zbuf, acc.at[pl.ds(pl.multiple_of(k * 64, 8), 64)])

                return carry2

            lax.fori_loop(0, 11, zero_body, None)
            plsc.subcore_barrier()

            def macro_body(m, carry2):
                row0 = pl.multiple_of(s * 64 + m * 8, 8)
                pltpu.sync_copy(sidx.at[rel, pl.ds(row0, 8)], sidx_v)
                pltpu.sync_copy(didx.at[p, pl.ds(row0, 8)], didx_v)

                def micro_body(k, carry3):
                    ja = 2 * k
                    jb = 2 * k + 1
                    slot0 = rows_v.at[pl.ds(0, 128)]
                    slot1 = rows_v.at[pl.ds(128, 128)]
                    g0 = pltpu.async_copy(tbl.at[sidx_v.at[ja]], slot0,
                                          sem_ga)
                    g1 = pltpu.async_copy(tbl.at[sidx_v.at[jb]], slot1,
                                          sem_gb)
                    g0.wait()
                    s0 = pltpu.async_copy(slot0, acc.at[didx_v.at[ja]],
                                          sem_sa, add=True)
                    g1.wait()
                    s1 = pltpu.async_copy(slot1, acc.at[didx_v.at[jb]],
                                          sem_sb, add=True)
                    s0.wait()
                    s1.wait()
                    return carry3

                lax.fori_loop(0, 4, micro_body, None)
                return carry2

            lax.fori_loop(0, 8, macro_body, None)
            plsc.subcore_barrier()

            def dump_body(i, carry2):
                k = s + 16 * i

                @pl.when(k < ndm)
                def _():
                    pltpu.sync_copy(
                        acc.at[pl.ds(pl.multiple_of(k * 800, 8), 800)],
                        out.at[pl.ds(
                            pl.multiple_of(out_base + k * 800, 8), 800)])

                return carry2

            lax.fori_loop(0, 1, dump_body, None)
            plsc.subcore_barrier()
            return carry

        lo = jnp.where(c == 0, 0, NPASS // 2)
        hi = jnp.where(c == 0, NPASS // 2, NPASS)
        lax.fori_loop(lo, hi, pass_body, None)

    return sc_agg


_sc_agg = _make_sc_agg()


def _l0_combine(sum0, x128, wcat, t):
    n_t = NN_T[t]
    rels = RELS_BY_DST[t]
    k = len(rels)

    def body(*refs):
        sum_refs = refs[:k]
        x_ref, w_ref, o_ref = refs[k], refs[k + 1], refs[k + 2]
        parts = []
        for sr_ref in sum_refs:
            sr = sr_ref[:, :16]
            invc = 1.0 / jnp.maximum(sr[:, 15:16], 1.0)
            parts.append(sr * invc)
        parts.append(x_ref[:, :16])
        a = jnp.concatenate(parts, axis=1)
        o_ref[...] = jnp.maximum(
            jax.lax.dot(a, w_ref[...], preferred_element_type=jnp.float32),
            0.0)

    in_specs = [
        pl.BlockSpec((BN, H), (lambda i, b=RELBASE[r] // BN: (b + i, 0)))
        for r in rels
    ]
    in_specs.append(
        pl.BlockSpec((BN, H), (lambda i, b=OFF_T[t] // BN: (b + i, 0))))
    in_specs.append(pl.BlockSpec((16 * (k + 1), H), lambda i: (0, 0)))
    return pl.pallas_call(
        body,
        grid=(n_t // BN,),
        in_specs=in_specs,
        out_specs=pl.BlockSpec((BN, H), lambda i: (i, 0)),
        out_shape=jax.ShapeDtypeStruct((n_t, H), jnp.float32),
    )(*([sum0] * k + [x128, wcat]))


def _l1_combine(sum1, sum0, ht, wls, m1, beta, t):
    n_t = NN_T[t]
    rels = RELS_BY_DST[t]
    k = len(rels)

    def body(*refs):
        agg_refs = refs[:k]
        cnt_refs = refs[k:2 * k]
        h_ref = refs[2 * k]
        wl_refs = refs[2 * k + 1:3 * k + 1]
        m1_ref, beta_ref, o_ref = refs[3 * k + 1:3 * k + 4]
        acc = jax.lax.dot(h_ref[...], m1_ref[...],
                          preferred_element_type=jnp.float32)
        for ri in range(k):
            invc = 1.0 / jnp.maximum(cnt_refs[ri][:, 15:16], 1.0)
            acc = acc + jax.lax.dot(agg_refs[ri][...] * invc, wl_refs[ri][...],
                                    preferred_element_type=jnp.float32)
        x = jnp.maximum(acc + beta_ref[...], 0.0)

        @pl.when(pl.program_id(0) == 0)
        def _():
            o_ref[...] = jnp.zeros_like(o_ref)

        o_ref[...] += jnp.sum(x, axis=0, keepdims=True)

    in_specs = []
    args = []
    for r in rels:
        in_specs.append(
            pl.BlockSpec((BN, H), (lambda i, b=RELBASE[r] // BN: (b + i, 0))))
        args.append(sum1)
    for r in rels:
        in_specs.append(
            pl.BlockSpec((BN, H), (lambda i, b=RELBASE[r] // BN: (b + i, 0))))
        args.append(sum0)
    in_specs.append(
        pl.BlockSpec((BN, H), (lambda i, b=OFF_T[t] // BN: (b + i, 0))))
    args.append(ht)
    for ri in range(k):
        in_specs.append(pl.BlockSpec((H, H), lambda i: (0, 0)))
        args.append(wls[ri])
    in_specs.append(pl.BlockSpec((H, H), lambda i: (0, 0)))
    args.append(m1)
    in_specs.append(pl.BlockSpec((1, H), lambda i: (0, 0)))
    args.append(beta)
    return pl.pallas_call(
        body,
        grid=(n_t // BN,),
        in_specs=in_specs,
        out_specs=pl.BlockSpec((1, H), lambda i: (0, 0)),
        out_shape=jax.ShapeDtypeStruct((1, H), jnp.float32),
    )(*args)


def _head(pools, w1, b1, w2, b2):
    def body(pv, pe, pf, w1r, b1r, w2r, b2r, o_ref):
        pooled = jnp.concatenate([pv[...], pe[...], pf[...]], axis=1)
        h = jnp.maximum(
            jax.lax.dot(pooled, w1r[...], preferred_element_type=jnp.float32)
            + b1r[...], 0.0)
        o_ref[...] = jax.lax.dot(
            h, w2r[...], preferred_element_type=jnp.float32) + b2r[...]

    return pl.pallas_call(
        body,
        out_shape=jax.ShapeDtypeStruct((1, 128), jnp.float32),
    )(pools[0], pools[1], pools[2], w1, b1, w2, b2)


def kernel(x_v, x_e, x_f, ei_vv, ei_ve, ei_vf, ei_ev, ei_ef, ei_fv, ei_fe,
           params):
    p = params
    eid = {'vv': ei_vv, 've': ei_ve, 'vf': ei_vf, 'ev': ei_ev, 'ef': ei_ef,
           'fv': ei_fv, 'fe': ei_fe}
    s = (1.0 + EPS) ** -0.5

    npad = EPAD - E
    pad_src = jnp.arange(npad, dtype=jnp.int32) % 128
    dummy = BS + (jnp.arange(EPAD, dtype=jnp.int32) % 64)
    sidx = jnp.stack([
        jnp.concatenate([eid[name][0] + OFF_T[st], pad_src])
        for name, st, dt in REL
    ]).reshape(7, 1024, 128)
    dst_pad = {
        r: jnp.concatenate(
            [eid[REL[r][0]][1],
             jnp.full((npad,), -1, jnp.int32)]) for r in range(7)
    }
    didx = jnp.stack([
        jnp.where((dst_pad[r] >= b * BS) & (dst_pad[r] < b * BS + BS),
                  dst_pad[r] - b * BS, dummy) for r, b, _rows in PASSES
    ]).reshape(NPASS, 1024, 128)

    def pad128(x):
        n, f = x.shape
        return jnp.concatenate(
            [x, jnp.zeros((n, 15 - f), jnp.float32),
             jnp.ones((n, 1), jnp.float32),
             jnp.zeros((n, 112), jnp.float32)], axis=1)

    x128 = jnp.concatenate([pad128(x_v), pad128(x_e), pad128(x_f)], axis=0)

    wcat0 = {}
    wl1p = {}
    m1f = {}
    beta1 = {}
    for t in ('v', 'e', 'f'):
        g0 = p['bn_l0_' + t + '_g']
        b0 = p['bn_l0_' + t + '_b']
        rels = RELS_BY_DST[t]
        K = len(rels)
        ft = FEAT0[t]
        parts = []
        for r in rels:
            name, st, _ = REL[r]
            wl = p['l0_' + name + '_Wl']
            wl16 = jnp.zeros((16, H), jnp.float32).at[:FEAT0[st]].set(wl)
            parts.append(wl16 * (s / K) * g0[None, :])
        wrm = sum(p['l0_' + REL[r][0] + '_Wr'] for r in rels) / K
        blm = sum(p['l0_' + REL[r][0] + '_bl'] for r in rels) / K
        m0 = jnp.zeros((16, H), jnp.float32)
        m0 = m0.at[:ft].set(s * wrm * g0[None, :])
        m0 = m0.at[15].set(s * blm * g0 + b0)
        wcat0[t] = jnp.concatenate(parts + [m0], axis=0)

        g1 = p['bn_l1_' + t + '_g']
        b1 = p['bn_l1_' + t + '_b']
        wl1p[t] = [
            p['l1_' + REL[r][0] + '_Wl'] * (s / K) * g1[None, :] for r in rels
        ]
        wr1m = sum(p['l1_' + REL[r][0] + '_Wr'] for r in rels) / K
        m1f[t] = s * (wr1m + jnp.eye(H, dtype=jnp.float32)) * g1[None, :]
        bl1m = sum(p['l1_' + REL[r][0] + '_bl'] for r in rels) / K
        beta1[t] = (s * bl1m * g1 + b1)[None, :]

    gf = p['fcbn_g']
    bf = p['fcbn_b']
    rowscale = jnp.concatenate([
        jnp.full((H,), 1.0 / NV, jnp.float32),
        jnp.full((H,), 1.0 / NE, jnp.float32),
        jnp.full((H,), 1.0 / NF, jnp.float32)])
    w1f = p['fc1_W'] * rowscale[:, None] * (s * gf)[None, :]
    b1f = (s * p['fc1_b'] * gf + bf)[None, :]
    w2p = jnp.zeros((256, 128), jnp.float32).at[:, :10].set(p['fc2_W'])
    b2p = jnp.zeros((1, 128), jnp.float32).at[0, :10].set(p['fc2_b'])

    zeros_h = jnp.zeros((64, H), jnp.float32)

    x128, sidx, didx, zeros_h = lax.optimization_barrier(
        (x128, sidx, didx, zeros_h))

    sum0 = _sc_agg(x128, sidx, didx, zeros_h)
    h0 = {t: _l0_combine(sum0, x128, wcat0[t], t) for t in ('v', 'e', 'f')}
    ht = jnp.concatenate([h0['v'], h0['e'], h0['f']], axis=0)

    ht = lax.optimization_barrier(ht)
    sum1 = _sc_agg(ht, sidx, didx, zeros_h)
    pools = [
        _l1_combine(sum1, sum0, ht, wl1p[t], m1f[t], beta1[t], t)
        for t in ('v', 'e', 'f')
    ]

    out = _head(pools, w1f, b1f, w2p, b2p)
    return out[0, :10]


# speedup vs baseline: 2.1494x; 1.0363x over previous
import functools

import jax
import jax.numpy as jnp
from jax import lax
from jax.experimental import pallas as pl
from jax.experimental.pallas import tpu as pltpu
from jax.experimental.pallas import tpu_sc as plsc

H = 128
EPS = 1e-5
NV, NE, NF = 20000, 60000, 40000
E = 128000
EPAD = 131072
REL = [('vv', 'v', 'v'), ('ve', 'v', 'e'), ('vf', 'v', 'f'), ('ev', 'e', 'v'),
       ('ef', 'e', 'f'), ('fv', 'f', 'v'), ('fe', 'f', 'e')]
FEAT0 = {'v': 7, 'e': 2, 'f': 5}
NN_T = {'v': NV, 'e': NE, 'f': NF}
OFF_T = {'v': 0, 'e': NV, 'f': NV + NE}
RELS_BY_DST = {'v': [0, 3, 5], 'e': [1, 6], 'f': [2, 4]}
N_DT = [20000, 60000, 40000, 20000, 40000, 20000, 60000]
BS = 10400
ACC_ROWS = BS + 64
BN = 2000


def _group_layout(rels):
    local_base = {}
    base = 0
    for r in rels:
        local_base[r] = base
        base += N_DT[r]
    passes = []
    for r in rels:
        n = N_DT[r]
        for b in range((n + BS - 1) // BS):
            passes.append((r, b, min(BS, n - b * BS)))
    return local_base, passes, base


def _sel(p, vals):
    out = jnp.int32(vals[-1])
    for i in range(len(vals) - 2, -1, -1):
        out = jnp.where(p == i, jnp.int32(vals[i]), out)
    return out


def _make_sc_agg(rels):
    local_base, passes, out_rows = _group_layout(rels)
    npass = len(passes)
    rel_of = [t[0] for t in passes]
    base_of = [local_base[t[0]] + BS * t[1] for t in passes]
    nzl_of = [(t[2] + 63) // 64 for t in passes]
    ndm_of = [t[2] // 800 for t in passes]
    mesh = plsc.VectorSubcoreMesh(core_axis_name="c", subcore_axis_name="s")

    @functools.partial(
        pl.kernel,
        out_type=jax.ShapeDtypeStruct((out_rows, H), jnp.float32),
        mesh=mesh,
        scratch_types=[
            pltpu.VMEM((8, 128), jnp.int32),
            pltpu.VMEM((8, 128), jnp.int32),
            pltpu.VMEM((256, H), jnp.float32),
            pltpu.VMEM((64, H), jnp.float32),
            pltpu.VMEM_SHARED((ACC_ROWS, H), jnp.float32),
            pltpu.SemaphoreType.DMA,
            pltpu.SemaphoreType.DMA,
            pltpu.SemaphoreType.DMA,
            pltpu.SemaphoreType.DMA,
        ],
    )
    def sc_agg(tbl, sidx, didx, zeros_h, out, sidx_v, didx_v, rows_v, zbuf,
               acc, sem_ga, sem_gb, sem_sa, sem_sb):
        c = lax.axis_index("c")
        s = lax.axis_index("s")
        pltpu.sync_copy(zeros_h, zbuf)

        def pass_body(p, carry):
            rel = _sel(p, rel_of)
            out_base = _sel(p, base_of)
            nzl = _sel(p, nzl_of)
            ndm = _sel(p, ndm_of)

            def zero_body(i, carry2):
                k = s + 16 * i

                @pl.when(k < nzl)
                def _():
                    pltpu.sync_copy(
                        zbuf, acc.at[pl.ds(pl.multiple_of(k * 64, 8), 64)])

                return carry2

            lax.fori_loop(0, 11, zero_body, None)
            plsc.subcore_barrier()

            def macro_body(m, carry2):
                row0 = pl.multiple_of(s * 64 + m * 8, 8)
                pltpu.sync_copy(sidx.at[rel, pl.ds(row0, 8)], sidx_v)
                pltpu.sync_copy(didx.at[p, pl.ds(row0, 8)], didx_v)

                def micro_body(k, carry3):
                    ja = 2 * k
                    jb = 2 * k + 1
                    slot0 = rows_v.at[pl.ds(0, 128)]
                    slot1 = rows_v.at[pl.ds(128, 128)]
                    g0 = pltpu.async_copy(tbl.at[sidx_v.at[ja]], slot0,
                                          sem_ga)
                    g1 = pltpu.async_copy(tbl.at[sidx_v.at[jb]], slot1,
                                          sem_gb)
                    g0.wait()
                    s0 = pltpu.async_copy(slot0, acc.at[didx_v.at[ja]],
                                          sem_sa, add=True)
                    g1.wait()
                    s1 = pltpu.async_copy(slot1, acc.at[didx_v.at[jb]],
                                          sem_sb, add=True)
                    s0.wait()
                    s1.wait()
                    return carry3

                lax.fori_loop(0, 4, micro_body, None)
                return carry2

            lax.fori_loop(0, 8, macro_body, None)
            plsc.subcore_barrier()

            def dump_body(i, carry2):
                k = s + 16 * i

                @pl.when(k < ndm)
                def _():
                    pltpu.sync_copy(
                        acc.at[pl.ds(pl.multiple_of(k * 800, 8), 800)],
                        out.at[pl.ds(
                            pl.multiple_of(out_base + k * 800, 8), 800)])

                return carry2

            lax.fori_loop(0, 1, dump_body, None)
            plsc.subcore_barrier()
            return carry

        lo = jnp.where(c == 0, 0, npass // 2)
        hi = jnp.where(c == 0, npass // 2, npass)
        lax.fori_loop(lo, hi, pass_body, None)

    return sc_agg


_sc_agg_t = {t: _make_sc_agg(RELS_BY_DST[t]) for t in ('v', 'e', 'f')}


def _l0_combine(sum0, x128, wcat, t):
    n_t = NN_T[t]
    rels = RELS_BY_DST[t]
    local_base, _, _ = _group_layout(rels)
    k = len(rels)

    def body(*refs):
        sum_refs = refs[:k]
        x_ref, w_ref, o_ref = refs[k], refs[k + 1], refs[k + 2]
        parts = []
        for sr_ref in sum_refs:
            sr = sr_ref[:, :16]
            invc = 1.0 / jnp.maximum(sr[:, 15:16], 1.0)
            parts.append(sr * invc)
        parts.append(x_ref[:, :16])
        a = jnp.concatenate(parts, axis=1)
        o_ref[...] = jnp.maximum(
            jax.lax.dot(a, w_ref[...], preferred_element_type=jnp.float32),
            0.0)

    in_specs = [
        pl.BlockSpec((BN, H), (lambda i, b=local_base[r] // BN: (b + i, 0)))
        for r in rels
    ]
    in_specs.append(
        pl.BlockSpec((BN, H), (lambda i, b=OFF_T[t] // BN: (b + i, 0))))
    in_specs.append(pl.BlockSpec((16 * (k + 1), H), lambda i: (0, 0)))
    return pl.pallas_call(
        body,
        grid=(n_t // BN,),
        in_specs=in_specs,
        out_specs=pl.BlockSpec((BN, H), lambda i: (i, 0)),
        out_shape=jax.ShapeDtypeStruct((n_t, H), jnp.float32),
    )(*([sum0] * k + [x128, wcat]))


def _l1_combine(sum1, sum0, ht, wls, m1, beta, t):
    n_t = NN_T[t]
    rels = RELS_BY_DST[t]
    local_base, _, _ = _group_layout(rels)
    k = len(rels)

    def body(*refs):
        agg_refs = refs[:k]
        cnt_refs = refs[k:2 * k]
        h_ref = refs[2 * k]
        wl_refs = refs[2 * k + 1:3 * k + 1]
        m1_ref, beta_ref, o_ref = refs[3 * k + 1:3 * k + 4]
        acc = jax.lax.dot(h_ref[...], m1_ref[...],
                          preferred_element_type=jnp.float32)
        for ri in range(k):
            invc = 1.0 / jnp.maximum(cnt_refs[ri][:, 15:16], 1.0)
            acc = acc + jax.lax.dot(agg_refs[ri][...] * invc, wl_refs[ri][...],
                                    preferred_element_type=jnp.float32)
        x = jnp.maximum(acc + beta_ref[...], 0.0)

        @pl.when(pl.program_id(0) == 0)
        def _():
            o_ref[...] = jnp.zeros_like(o_ref)

        o_ref[...] += jnp.sum(x, axis=0, keepdims=True)

    in_specs = []
    args = []
    for r in rels:
        in_specs.append(
            pl.BlockSpec((BN, H),
                         (lambda i, b=local_base[r] // BN: (b + i, 0))))
        args.append(sum1)
    for r in rels:
        in_specs.append(
            pl.BlockSpec((BN, H),
                         (lambda i, b=local_base[r] // BN: (b + i, 0))))
        args.append(sum0)
    in_specs.append(
        pl.BlockSpec((BN, H), (lambda i, b=OFF_T[t] // BN: (b + i, 0))))
    args.append(ht)
    for ri in range(k):
        in_specs.append(pl.BlockSpec((H, H), lambda i: (0, 0)))
        args.append(wls[ri])
    in_specs.append(pl.BlockSpec((H, H), lambda i: (0, 0)))
    args.append(m1)
    in_specs.append(pl.BlockSpec((1, H), lambda i: (0, 0)))
    args.append(beta)
    return pl.pallas_call(
        body,
        grid=(n_t // BN,),
        in_specs=in_specs,
        out_specs=pl.BlockSpec((1, H), lambda i: (0, 0)),
        out_shape=jax.ShapeDtypeStruct((1, H), jnp.float32),
    )(*args)


def _head(pools, w1, b1, w2, b2):
    def body(pv, pe, pf, w1r, b1r, w2r, b2r, o_ref):
        pooled = jnp.concatenate([pv[...], pe[...], pf[...]], axis=1)
        h = jnp.maximum(
            jax.lax.dot(pooled, w1r[...], preferred_element_type=jnp.float32)
            + b1r[...], 0.0)
        o_ref[...] = jax.lax.dot(
            h, w2r[...], preferred_element_type=jnp.float32) + b2r[...]

    return pl.pallas_call(
        body,
        out_shape=jax.ShapeDtypeStruct((1, 128), jnp.float32),
    )(pools[0], pools[1], pools[2], w1, b1, w2, b2)


def kernel(x_v, x_e, x_f, ei_vv, ei_ve, ei_vf, ei_ev, ei_ef, ei_fv, ei_fe,
           params):
    p = params
    eid = {'vv': ei_vv, 've': ei_ve, 'vf': ei_vf, 'ev': ei_ev, 'ef': ei_ef,
           'fv': ei_fv, 'fe': ei_fe}
    s = (1.0 + EPS) ** -0.5

    npad = EPAD - E
    pad_src = jnp.arange(npad, dtype=jnp.int32) % 128
    dummy = BS + (jnp.arange(EPAD, dtype=jnp.int32) % 64)
    sidx = jnp.stack([
        jnp.concatenate([eid[name][0] + OFF_T[st], pad_src])
        for name, st, dt in REL
    ]).reshape(7, 1024, 128)
    dst_pad = {
        r: jnp.concatenate(
            [eid[REL[r][0]][1],
             jnp.full((npad,), -1, jnp.int32)]) for r in range(7)
    }
    didx_t = {}
    for t in ('v', 'e', 'f'):
        _, passes, _ = _group_layout(RELS_BY_DST[t])
        didx_t[t] = jnp.stack([
            jnp.where((dst_pad[r] >= b * BS) & (dst_pad[r] < b * BS + BS),
                      dst_pad[r] - b * BS, dummy) for r, b, _rows in passes
        ]).reshape(len(passes), 1024, 128)

    def pad128(x):
        n, f = x.shape
        return jnp.concatenate(
            [x, jnp.zeros((n, 15 - f), jnp.float32),
             jnp.ones((n, 1), jnp.float32),
             jnp.zeros((n, 112), jnp.float32)], axis=1)

    x128 = jnp.concatenate([pad128(x_v), pad128(x_e), pad128(x_f)], axis=0)

    wcat0 = {}
    wl1p = {}
    m1f = {}
    beta1 = {}
    for t in ('v', 'e', 'f'):
        g0 = p['bn_l0_' + t + '_g']
        b0 = p['bn_l0_' + t + '_b']
        rels = RELS_BY_DST[t]
        K = len(rels)
        ft = FEAT0[t]
        parts = []
        for r in rels:
            name, st, _ = REL[r]
            wl = p['l0_' + name + '_Wl']
            wl16 = jnp.zeros((16, H), jnp.float32).at[:FEAT0[st]].set(wl)
            parts.append(wl16 * (s / K) * g0[None, :])
        wrm = sum(p['l0_' + REL[r][0] + '_Wr'] for r in rels) / K
        blm = sum(p['l0_' + REL[r][0] + '_bl'] for r in rels) / K
        m0 = jnp.zeros((16, H), jnp.float32)
        m0 = m0.at[:ft].set(s * wrm * g0[None, :])
        m0 = m0.at[15].set(s * blm * g0 + b0)
        wcat0[t] = jnp.concatenate(parts + [m0], axis=0)

        g1 = p['bn_l1_' + t + '_g']
        b1 = p['bn_l1_' + t + '_b']
        wl1p[t] = [
            p['l1_' + REL[r][0] + '_Wl'] * (s / K) * g1[None, :] for r in rels
        ]
        wr1m = sum(p['l1_' + REL[r][0] + '_Wr'] for r in rels) / K
        m1f[t] = s * (wr1m + jnp.eye(H, dtype=jnp.float32)) * g1[None, :]
        bl1m = sum(p['l1_' + REL[r][0] + '_bl'] for r in rels) / K
        beta1[t] = (s * bl1m * g1 + b1)[None, :]

    gf = p['fcbn_g']
    bf = p['fcbn_b']
    rowscale = jnp.concatenate([
        jnp.full((H,), 1.0 / NV, jnp.float32),
        jnp.full((H,), 1.0 / NE, jnp.float32),
        jnp.full((H,), 1.0 / NF, jnp.float32)])
    w1f = p['fc1_W'] * rowscale[:, None] * (s * gf)[None, :]
    b1f = (s * p['fc1_b'] * gf + bf)[None, :]
    w2p = jnp.zeros((256, 128), jnp.float32).at[:, :10].set(p['fc2_W'])
    b2p = jnp.zeros((1, 128), jnp.float32).at[0, :10].set(p['fc2_b'])

    zeros_h = jnp.zeros((64, H), jnp.float32)

    x128, sidx, zeros_h = lax.optimization_barrier((x128, sidx, zeros_h))
    didx_t = lax.optimization_barrier(didx_t)

    sum0 = {
        t: _sc_agg_t[t](x128, sidx, didx_t[t], zeros_h)
        for t in ('v', 'e', 'f')
    }
    h0 = {
        t: _l0_combine(sum0[t], x128, wcat0[t], t) for t in ('v', 'e', 'f')
    }
    ht = jnp.concatenate([h0['v'], h0['e'], h0['f']], axis=0)

    ht = lax.optimization_barrier(ht)
    sum1 = {
        t: _sc_agg_t[t](ht, sidx, didx_t[t], zeros_h)
        for t in ('v', 'e', 'f')
    }
    pools = [
        _l1_combine(sum1[t], sum0[t], ht, wl1p[t], m1f[t], beta1[t], t)
        for t in ('v', 'e', 'f')
    ]

    out = _head(pools, w1f, b1f, w2p, b2p)
    return out[0, :10]
